# baseline (speedup 1.0000x reference)
import functools

import jax
import jax.numpy as jnp
from jax import lax
from jax.experimental import pallas as pl
from jax.experimental.pallas import tpu as pltpu
from jax.experimental.pallas import tpu_sc as plsc

TOP_K = 70
INV_TEMP = 25.0

Q = 1024
D = 256
K = 100000
BK = 1024
KP = 100352
NB = KP // BK
CH = 128
NCH = KP // CH
CAND = 512
H = 256
NWORK = 32
QPW = Q // NWORK
TP = 80

_NEG = -1e9

_GDN = lax.GatherDimensionNumbers(
    offset_dims=(), collapsed_slice_dims=(0,), start_index_map=(0,))


def _bcast(v, i):
    idx = jnp.full((16, 1), i, jnp.int32)
    return lax.gather(v, idx, _GDN, (1,),
                      mode=lax.GatherScatterMode.PROMISE_IN_BOUNDS)


def _sims_body(q_ref, db_ref, sims_ref, cm_ref):
    j = pl.program_id(0)
    q = q_ref[...]
    qn = q / jnp.maximum(jnp.sqrt(jnp.sum(q * q, axis=1, keepdims=True)), 1e-12)
    db = db_ref[...]
    dbn = db / jnp.maximum(jnp.sqrt(jnp.sum(db * db, axis=1, keepdims=True)), 1e-12)
    sims = lax.dot_general(
        qn, dbn, (((1,), (1,)), ((), ())), preferred_element_type=jnp.float32)
    col = j * BK + lax.broadcasted_iota(jnp.int32, (Q, BK), 1)
    sims = jnp.where(col < K, sims, -2.0)
    sims_ref[...] = sims
    for i in range(BK // CH):
        cm_ref[i, :] = jnp.max(sims[:, i * CH:(i + 1) * CH], axis=1)


@jax.jit
def _stage_a(query, en_db):
    en_pad = jnp.pad(en_db, ((0, KP - K), (0, 0)))
    return pl.pallas_call(
        _sims_body,
        grid=(NB,),
        in_specs=[
            pl.BlockSpec((Q, D), lambda j: (0, 0)),
            pl.BlockSpec((BK, D), lambda j: (j, 0)),
        ],
        out_specs=[
            pl.BlockSpec((Q, BK), lambda j: (0, j)),
            pl.BlockSpec((BK // CH, Q), lambda j: (j, 0)),
        ],
        out_shape=[
            jax.ShapeDtypeStruct((Q, KP), jnp.float32),
            jax.ShapeDtypeStruct((NCH, Q), jnp.float32),
        ],
    )(query, en_pad)


def _tcm_body(cm_ref, t_ref, cmt_ref):
    x = cm_ref[...]
    cmt_ref[...] = x.T

    def step(_, x):
        m = jnp.max(x, axis=0, keepdims=True)
        return jnp.where(x >= m, -3.0, x)

    x = lax.fori_loop(0, TOP_K - 1, step, x)
    t_ref[...] = jnp.max(x, axis=0, keepdims=True)


@jax.jit
def _stage_b(cm):
    return pl.pallas_call(
        _tcm_body,
        out_shape=[
            jax.ShapeDtypeStruct((1, Q), jnp.float32),
            jax.ShapeDtypeStruct((Q, NCH), jnp.float32),
        ],
    )(cm)


def _compact_body(simsc, cmt, tcm, ovals, oidx,
                  tloc, cmloc, hitidx, hitbase, nh, gbuf0, gbuf1,
                  cvals, cidx, semg0, semg1):
    wid = lax.axis_index("s") * 2 + lax.axis_index("c")
    qbase = wid * QPW
    pltpu.sync_copy(tcm.at[0, pl.ds(qbase, QPW)], tloc)
    lane = lax.iota(jnp.int32, 16)

    def build(ql, _):
        qg = qbase + ql
        tq = plsc.load_gather(tloc, [jnp.full((16,), ql, jnp.int32)])
        pltpu.sync_copy(cmt.at[pl.ds(qg, 1)], cmloc)

        def initb(i, _):
            ids = i * 16 + lane
            hitidx[pl.ds(ql * H + i * 16, 16)] = qg * NCH + ids
            return 0

        lax.fori_loop(0, H // 16, initb, 0)

        def vstep(i, off):
            v = cmloc[0, pl.ds(i * 16, 16)]
            m = v >= tq
            c = plsc.cumsum(m.astype(jnp.int32))
            pos = ql * H + jnp.minimum(off + c - 1, H - 1)
            ids = i * 16 + lane
            plsc.store_scatter(hitidx, [pos], qg * NCH + ids, mask=m)
            plsc.store_scatter(hitbase, [pos], ids * CH, mask=m)
            return off + jnp.sum(m.astype(jnp.int32))

        off = lax.fori_loop(0, NCH // 16, vstep, jnp.int32(0))
        nh[ql] = jnp.minimum(off, H)
        return 0

    lax.fori_loop(0, QPW, build, 0)

    pltpu.async_copy(simsc.at[hitidx.at[pl.ds(0, H)]], gbuf0, semg0)

    def scan_query(ql, gbuf):
        qg = qbase + ql
        tq = plsc.load_gather(tloc, [jnp.full((16,), ql, jnp.int32)])

        def initc(i, _):
            cvals[pl.ds(i * 16, 16)] = jnp.full((16,), _NEG, jnp.float32)
            return 0

        lax.fori_loop(0, CAND // 16, initc, 0)
        n = nh[ql]

        def chunk(j, off):
            hb = hitbase[pl.ds(ql * H + (j & ~15), 16)]
            kbase = _bcast(hb, j & 15)
            for r in range(8):
                v = gbuf[j, pl.ds(r * 16, 16)]
                m = v >= tq
                c = plsc.cumsum(m.astype(jnp.int32))
                pos = jnp.minimum(off + c - 1, CAND - 1)
                idxv = kbase + (r * 16 + lane)
                plsc.store_scatter(cvals, [pos], v, mask=m)
                plsc.store_scatter(cidx, [pos], idxv, mask=m)
                off = off + jnp.sum(m.astype(jnp.int32))
            return off

        lax.fori_loop(0, n, chunk, jnp.int32(0))
        pltpu.sync_copy(cvals, ovals.at[qg])
        pltpu.sync_copy(cidx, oidx.at[qg])

    def outer(o, _):
        for b, (gbuf, semg) in enumerate(((gbuf0, semg0), (gbuf1, semg1))):
            ql = o * 2 + b
            ngbuf = gbuf1 if b == 0 else gbuf0
            nsemg = semg1 if b == 0 else semg0

            pltpu.make_async_copy(simsc.at[hitidx.at[pl.ds(ql * H, H)]], gbuf, semg).wait()

            @pl.when(ql + 1 < QPW)
            def _():
                pltpu.async_copy(
                    simsc.at[hitidx.at[pl.ds((ql + 1) * H, H)]], ngbuf, nsemg)

            scan_query(ql, gbuf)
        return 0

    lax.fori_loop(0, QPW // 2, outer, 0)


@jax.jit
def _stage_c(sims, cmt, tcm):
    simsc = jnp.reshape(sims, (Q * NCH, CH))
    mesh = plsc.VectorSubcoreMesh(core_axis_name="c", subcore_axis_name="s")
    return pl.kernel(
        _compact_body,
        out_type=[
            jax.ShapeDtypeStruct((Q, CAND), jnp.float32),
            jax.ShapeDtypeStruct((Q, CAND), jnp.int32),
        ],
        mesh=mesh,
        compiler_params=pltpu.CompilerParams(needs_layout_passes=False),
        scratch_types=[
            pltpu.VMEM((QPW,), jnp.float32),
            pltpu.VMEM((1, NCH), jnp.float32),
            pltpu.VMEM((QPW * H,), jnp.int32),
            pltpu.VMEM((QPW * H,), jnp.int32),
            pltpu.SMEM((QPW,), jnp.int32),
            pltpu.VMEM((H, CH), jnp.float32),
            pltpu.VMEM((H, CH), jnp.float32),
            pltpu.VMEM((CAND,), jnp.float32),
            pltpu.VMEM((CAND,), jnp.int32),
            pltpu.SemaphoreType.DMA,
            pltpu.SemaphoreType.DMA,
        ],
    )(simsc, cmt, tcm)


def _select_body(vals_ref, idx_ref, w_ref, ti_ref):
    V = vals_ref[...]
    I = idx_ref[...]
    iota_c = lax.broadcasted_iota(jnp.int32, (Q, CAND), 1)
    iota_t = lax.broadcasted_iota(jnp.int32, (Q, TP), 1)

    def step(i, carry):
        V, tv, ti = carry
        m = jnp.max(V, axis=1, keepdims=True)
        eq = V >= m
        fidx = jnp.min(jnp.where(eq, iota_c, CAND), axis=1, keepdims=True)
        sel = iota_c == fidx
        selidx = jnp.sum(jnp.where(sel, I, 0), axis=1, keepdims=True)
        V = jnp.where(sel, _NEG, V)
        hit = iota_t == i
        tv = jnp.where(hit, m, tv)
        ti = jnp.where(hit, selidx, ti)
        return V, tv, ti

    tv0 = jnp.full((Q, TP), _NEG, jnp.float32)
    ti0 = jnp.zeros((Q, TP), jnp.int32)
    _, tv, ti = lax.fori_loop(0, TOP_K, step, (V, tv0, ti0))
    m0 = tv[:, 0:1]
    valid = iota_t < TOP_K
    e = jnp.where(valid, jnp.exp((tv - m0) * INV_TEMP), 0.0)
    z = jnp.sum(e, axis=1, keepdims=True)
    w_ref[...] = e / z
    ti_ref[...] = jnp.where(valid, ti, 0)


@jax.jit
def _stage_d(cvals, cidx):
    return pl.pallas_call(
        _select_body,
        out_shape=[
            jax.ShapeDtypeStruct((Q, TP), jnp.float32),
            jax.ShapeDtypeStruct((Q, TP), jnp.int32),
        ],
    )(cvals, cidx)


def _pool_body(ti, w, es, out, idxall, wall, idxb0, idxb1,
               rows0, rows1, obuf, semg0, semg1):
    wid = lax.axis_index("s") * 2 + lax.axis_index("c")
    qbase = wid * QPW
    pltpu.sync_copy(ti.at[pl.ds(qbase, QPW)], idxall)
    pltpu.sync_copy(w.at[pl.ds(qbase, QPW)], wall)

    def fill_idx(ql, idxb):
        for i in range(TP // 16):
            idxb[pl.ds(i * 16, 16)] = idxall[ql, pl.ds(i * 16, 16)]

    fill_idx(0, idxb0)
    pltpu.async_copy(es.at[idxb0], rows0, semg0)

    def pool_query(ql, rows):
        qg = qbase + ql
        for dg in range(4):

            def jbstep(jb, accs):
                wv = wall[ql, pl.ds(jb * 16, 16)]
                for r in range(16):
                    wj = _bcast(wv, r)
                    j = jb * 16 + r
                    accs = tuple(
                        accs[d] + wj * rows[j, pl.ds((dg * 4 + d) * 16, 16)]
                        for d in range(4))
                return accs

            accs = tuple(jnp.zeros((16,), jnp.float32) for _ in range(4))
            accs = lax.fori_loop(0, TP // 16, jbstep, accs)
            for d in range(4):
                obuf[pl.ds((dg * 4 + d) * 16, 16)] = accs[d]
        pltpu.sync_copy(obuf, out.at[qg])

    def outer(o, _):
        for b, (rows, semg) in enumerate(((rows0, semg0), (rows1, semg1))):
            ql = o * 2 + b
            nrows = rows1 if b == 0 else rows0
            nsemg = semg1 if b == 0 else semg0

            idxb = idxb0 if b == 0 else idxb1
            nidxb = idxb1 if b == 0 else idxb0

            pltpu.make_async_copy(es.at[idxb], rows, semg).wait()

            @pl.when(ql + 1 < QPW)
            def _():
                fill_idx(ql + 1, nidxb)
                pltpu.async_copy(es.at[nidxb], nrows, nsemg)

            pool_query(ql, rows)
        return 0

    lax.fori_loop(0, QPW // 2, outer, 0)


@jax.jit
def _stage_e(ti, w, es_db):
    mesh = plsc.VectorSubcoreMesh(core_axis_name="c", subcore_axis_name="s")
    return pl.kernel(
        _pool_body,
        out_type=jax.ShapeDtypeStruct((Q, D), jnp.float32),
        mesh=mesh,
        compiler_params=pltpu.CompilerParams(needs_layout_passes=False),
        scratch_types=[
            pltpu.VMEM((QPW, TP), jnp.int32),
            pltpu.VMEM((QPW, TP), jnp.float32),
            pltpu.VMEM((TP,), jnp.int32),
            pltpu.VMEM((TP,), jnp.int32),
            pltpu.VMEM((TP, D), jnp.float32),
            pltpu.VMEM((TP, D), jnp.float32),
            pltpu.VMEM((D,), jnp.float32),
            pltpu.SemaphoreType.DMA,
            pltpu.SemaphoreType.DMA,
        ],
    )(ti, w, es_db)


def kernel(query, en_db, es_db):
    sims, cm = _stage_a(query, en_db)
    tcm, cmt = _stage_b(cm)
    cvals, cidx = _stage_c(sims, cmt, tcm)
    w, ti = _stage_d(cvals, cidx)
    return _stage_e(ti, w, es_db)


# speedup vs baseline: 1.0198x; 1.0198x over previous
import functools

import jax
import jax.numpy as jnp
from jax import lax
from jax.experimental import pallas as pl
from jax.experimental.pallas import tpu as pltpu
from jax.experimental.pallas import tpu_sc as plsc

TOP_K = 70
INV_TEMP = 25.0

Q = 1024
D = 256
K = 100000
BK = 1024
KP = 100352
NB = KP // BK
CH = 128
NCH = KP // CH
CAND = 512
H = 256
NWORK = 32
QPW = Q // NWORK
TP = 80

_NEG = -1e9

_GDN = lax.GatherDimensionNumbers(
    offset_dims=(), collapsed_slice_dims=(0,), start_index_map=(0,))


def _bcast(v, i):
    idx = jnp.full((16, 1), i, jnp.int32)
    return lax.gather(v, idx, _GDN, (1,),
                      mode=lax.GatherScatterMode.PROMISE_IN_BOUNDS)


def _sims_body(q_ref, db_ref, sims_ref, t_ref, cmt_ref, qn_ref, cm_ref):
    j = pl.program_id(0)

    @pl.when(j == 0)
    def _():
        q = q_ref[...]
        qn_ref[...] = q / jnp.maximum(
            jnp.sqrt(jnp.sum(q * q, axis=1, keepdims=True)), 1e-12)

    qn = qn_ref[...]
    db = db_ref[...]
    dbn = db / jnp.maximum(jnp.sqrt(jnp.sum(db * db, axis=1, keepdims=True)), 1e-12)
    sims = lax.dot_general(
        qn, dbn, (((1,), (1,)), ((), ())), preferred_element_type=jnp.float32)
    col = j * BK + lax.broadcasted_iota(jnp.int32, (Q, BK), 1)
    sims = jnp.where(col < K, sims, -2.0)
    sims_ref[...] = sims
    for i in range(BK // CH):
        cm_ref[j * (BK // CH) + i, :] = jnp.max(
            sims[:, i * CH:(i + 1) * CH], axis=1)

    @pl.when(j == NB - 1)
    def _():
        x = cm_ref[...]
        cmt_ref[...] = x.T

        def step(_, x):
            m = jnp.max(x, axis=0, keepdims=True)
            return jnp.where(x >= m, -3.0, x)

        x = lax.fori_loop(0, TOP_K - 1, step, x)
        t_ref[...] = jnp.max(x, axis=0, keepdims=True)


@jax.jit
def _stage_a(query, en_db):
    en_pad = jnp.pad(en_db, ((0, KP - K), (0, 0)))
    return pl.pallas_call(
        _sims_body,
        grid=(NB,),
        in_specs=[
            pl.BlockSpec((Q, D), lambda j: (0, 0)),
            pl.BlockSpec((BK, D), lambda j: (j, 0)),
        ],
        out_specs=[
            pl.BlockSpec((Q, BK), lambda j: (0, j)),
            pl.BlockSpec((1, Q), lambda j: (0, 0)),
            pl.BlockSpec((Q, NCH), lambda j: (0, 0)),
        ],
        out_shape=[
            jax.ShapeDtypeStruct((Q, KP), jnp.float32),
            jax.ShapeDtypeStruct((1, Q), jnp.float32),
            jax.ShapeDtypeStruct((Q, NCH), jnp.float32),
        ],
        scratch_shapes=[
            pltpu.VMEM((Q, D), jnp.float32),
            pltpu.VMEM((NCH, Q), jnp.float32),
        ],
    )(query, en_pad)


def _tcm_body(cm_ref, t_ref, cmt_ref):
    x = cm_ref[...]
    cmt_ref[...] = x.T

    def step(_, x):
        m = jnp.max(x, axis=0, keepdims=True)
        return jnp.where(x >= m, -3.0, x)

    x = lax.fori_loop(0, TOP_K - 1, step, x)
    t_ref[...] = jnp.max(x, axis=0, keepdims=True)


@jax.jit
def _stage_b(cm):
    return pl.pallas_call(
        _tcm_body,
        out_shape=[
            jax.ShapeDtypeStruct((1, Q), jnp.float32),
            jax.ShapeDtypeStruct((Q, NCH), jnp.float32),
        ],
    )(cm)


def _compact_body(simsc, cmt, tcm, ovals, oidx,
                  tloc, cmloc, hitidx, hitbase, nh, gbuf0, gbuf1,
                  cvals, cidx, semg0, semg1):
    wid = lax.axis_index("s") * 2 + lax.axis_index("c")
    qbase = wid * QPW
    pltpu.sync_copy(tcm.at[0, pl.ds(qbase, QPW)], tloc)
    lane = lax.iota(jnp.int32, 16)

    def build(ql, _):
        qg = qbase + ql
        tq = plsc.load_gather(tloc, [jnp.full((16,), ql, jnp.int32)])
        pltpu.sync_copy(cmt.at[pl.ds(qg, 1)], cmloc)

        def initb(i, _):
            ids = i * 16 + lane
            hitidx[pl.ds(ql * H + i * 16, 16)] = qg * NCH + ids
            return 0

        lax.fori_loop(0, H // 16, initb, 0)

        def vstep(i, off):
            v = cmloc[0, pl.ds(i * 16, 16)]
            m = v >= tq
            c = plsc.cumsum(m.astype(jnp.int32))
            pos = ql * H + jnp.minimum(off + c - 1, H - 1)
            ids = i * 16 + lane
            plsc.store_scatter(hitidx, [pos], qg * NCH + ids, mask=m)
            plsc.store_scatter(hitbase, [pos], ids * CH, mask=m)
            return off + jnp.sum(m.astype(jnp.int32))

        off = lax.fori_loop(0, NCH // 16, vstep, jnp.int32(0))
        nh[ql] = jnp.minimum(off, H)
        return 0

    lax.fori_loop(0, QPW, build, 0)

    pltpu.async_copy(simsc.at[hitidx.at[pl.ds(0, H)]], gbuf0, semg0)

    def scan_query(ql, gbuf):
        qg = qbase + ql
        tq = plsc.load_gather(tloc, [jnp.full((16,), ql, jnp.int32)])

        def initc(i, _):
            cvals[pl.ds(i * 16, 16)] = jnp.full((16,), _NEG, jnp.float32)
            return 0

        lax.fori_loop(0, CAND // 16, initc, 0)
        n = nh[ql]

        def chunk(j, off):
            hb = hitbase[pl.ds(ql * H + (j & ~15), 16)]
            kbase = _bcast(hb, j & 15)
            for r in range(8):
                v = gbuf[j, pl.ds(r * 16, 16)]
                m = v >= tq
                c = plsc.cumsum(m.astype(jnp.int32))
                pos = jnp.minimum(off + c - 1, CAND - 1)
                idxv = kbase + (r * 16 + lane)
                plsc.store_scatter(cvals, [pos], v, mask=m)
                plsc.store_scatter(cidx, [pos], idxv, mask=m)
                off = off + jnp.sum(m.astype(jnp.int32))
            return off

        lax.fori_loop(0, n, chunk, jnp.int32(0))
        pltpu.sync_copy(cvals, ovals.at[qg])
        pltpu.sync_copy(cidx, oidx.at[qg])

    def outer(o, _):
        for b, (gbuf, semg) in enumerate(((gbuf0, semg0), (gbuf1, semg1))):
            ql = o * 2 + b
            ngbuf = gbuf1 if b == 0 else gbuf0
            nsemg = semg1 if b == 0 else semg0

            pltpu.make_async_copy(simsc.at[hitidx.at[pl.ds(ql * H, H)]], gbuf, semg).wait()

            @pl.when(ql + 1 < QPW)
            def _():
                pltpu.async_copy(
                    simsc.at[hitidx.at[pl.ds((ql + 1) * H, H)]], ngbuf, nsemg)

            scan_query(ql, gbuf)
        return 0

    lax.fori_loop(0, QPW // 2, outer, 0)


@jax.jit
def _stage_c(sims, cmt, tcm):
    simsc = jnp.reshape(sims, (Q * NCH, CH))
    mesh = plsc.VectorSubcoreMesh(core_axis_name="c", subcore_axis_name="s")
    return pl.kernel(
        _compact_body,
        out_type=[
            jax.ShapeDtypeStruct((Q, CAND), jnp.float32),
            jax.ShapeDtypeStruct((Q, CAND), jnp.int32),
        ],
        mesh=mesh,
        compiler_params=pltpu.CompilerParams(needs_layout_passes=False),
        scratch_types=[
            pltpu.VMEM((QPW,), jnp.float32),
            pltpu.VMEM((1, NCH), jnp.float32),
            pltpu.VMEM((QPW * H,), jnp.int32),
            pltpu.VMEM((QPW * H,), jnp.int32),
            pltpu.SMEM((QPW,), jnp.int32),
            pltpu.VMEM((H, CH), jnp.float32),
            pltpu.VMEM((H, CH), jnp.float32),
            pltpu.VMEM((CAND,), jnp.float32),
            pltpu.VMEM((CAND,), jnp.int32),
            pltpu.SemaphoreType.DMA,
            pltpu.SemaphoreType.DMA,
        ],
    )(simsc, cmt, tcm)


def _select_body(vals_ref, idx_ref, w_ref, ti_ref):
    V = vals_ref[...]
    I = idx_ref[...]
    iota_c = lax.broadcasted_iota(jnp.int32, (Q, CAND), 1)
    iota_t = lax.broadcasted_iota(jnp.int32, (Q, TP), 1)

    def step(i, carry):
        V, tv, ti = carry
        m = jnp.max(V, axis=1, keepdims=True)
        eq = V >= m
        fidx = jnp.min(jnp.where(eq, iota_c, CAND), axis=1, keepdims=True)
        sel = iota_c == fidx
        selidx = jnp.sum(jnp.where(sel, I, 0), axis=1, keepdims=True)
        V = jnp.where(sel, _NEG, V)
        hit = iota_t == i
        tv = jnp.where(hit, m, tv)
        ti = jnp.where(hit, selidx, ti)
        return V, tv, ti

    tv0 = jnp.full((Q, TP), _NEG, jnp.float32)
    ti0 = jnp.zeros((Q, TP), jnp.int32)
    _, tv, ti = lax.fori_loop(0, TOP_K, step, (V, tv0, ti0))
    m0 = tv[:, 0:1]
    valid = iota_t < TOP_K
    e = jnp.where(valid, jnp.exp((tv - m0) * INV_TEMP), 0.0)
    z = jnp.sum(e, axis=1, keepdims=True)
    w_ref[...] = e / z
    ti_ref[...] = jnp.where(valid, ti, 0)


@jax.jit
def _stage_d(cvals, cidx):
    return pl.pallas_call(
        _select_body,
        out_shape=[
            jax.ShapeDtypeStruct((Q, TP), jnp.float32),
            jax.ShapeDtypeStruct((Q, TP), jnp.int32),
        ],
    )(cvals, cidx)


def _pool_body(ti, w, es, out, idxall, wall, idxb0, idxb1,
               rows0, rows1, obuf, semg0, semg1):
    wid = lax.axis_index("s") * 2 + lax.axis_index("c")
    qbase = wid * QPW
    pltpu.sync_copy(ti.at[pl.ds(qbase, QPW)], idxall)
    pltpu.sync_copy(w.at[pl.ds(qbase, QPW)], wall)

    def fill_idx(ql, idxb):
        for i in range(TP // 16):
            idxb[pl.ds(i * 16, 16)] = idxall[ql, pl.ds(i * 16, 16)]

    fill_idx(0, idxb0)
    pltpu.async_copy(es.at[idxb0], rows0, semg0)

    def pool_query(ql, rows):
        qg = qbase + ql
        for dg in range(4):

            def jbstep(jb, accs):
                wv = wall[ql, pl.ds(jb * 16, 16)]
                for r in range(16):
                    wj = _bcast(wv, r)
                    j = jb * 16 + r
                    accs = tuple(
                        accs[d] + wj * rows[j, pl.ds((dg * 4 + d) * 16, 16)]
                        for d in range(4))
                return accs

            accs = tuple(jnp.zeros((16,), jnp.float32) for _ in range(4))
            accs = lax.fori_loop(0, TP // 16, jbstep, accs)
            for d in range(4):
                obuf[pl.ds((dg * 4 + d) * 16, 16)] = accs[d]
        pltpu.sync_copy(obuf, out.at[qg])

    def outer(o, _):
        for b, (rows, semg) in enumerate(((rows0, semg0), (rows1, semg1))):
            ql = o * 2 + b
            nrows = rows1 if b == 0 else rows0
            nsemg = semg1 if b == 0 else semg0

            idxb = idxb0 if b == 0 else idxb1
            nidxb = idxb1 if b == 0 else idxb0

            pltpu.make_async_copy(es.at[idxb], rows, semg).wait()

            @pl.when(ql + 1 < QPW)
            def _():
                fill_idx(ql + 1, nidxb)
                pltpu.async_copy(es.at[nidxb], nrows, nsemg)

            pool_query(ql, rows)
        return 0

    lax.fori_loop(0, QPW // 2, outer, 0)


@jax.jit
def _stage_e(ti, w, es_db):
    mesh = plsc.VectorSubcoreMesh(core_axis_name="c", subcore_axis_name="s")
    return pl.kernel(
        _pool_body,
        out_type=jax.ShapeDtypeStruct((Q, D), jnp.float32),
        mesh=mesh,
        compiler_params=pltpu.CompilerParams(needs_layout_passes=False),
        scratch_types=[
            pltpu.VMEM((QPW, TP), jnp.int32),
            pltpu.VMEM((QPW, TP), jnp.float32),
            pltpu.VMEM((TP,), jnp.int32),
            pltpu.VMEM((TP,), jnp.int32),
            pltpu.VMEM((TP, D), jnp.float32),
            pltpu.VMEM((TP, D), jnp.float32),
            pltpu.VMEM((D,), jnp.float32),
            pltpu.SemaphoreType.DMA,
            pltpu.SemaphoreType.DMA,
        ],
    )(ti, w, es_db)


def kernel(query, en_db, es_db):
    sims, tcm, cmt = _stage_a(query, en_db)
    cvals, cidx = _stage_c(sims, cmt, tcm)
    w, ti = _stage_d(cvals, cidx)
    return _stage_e(ti, w, es_db)


# speedup vs baseline: 1.0693x; 1.0486x over previous
import functools

import jax
import jax.numpy as jnp
from jax import lax
from jax.experimental import pallas as pl
from jax.experimental.pallas import tpu as pltpu
from jax.experimental.pallas import tpu_sc as plsc

TOP_K = 70
INV_TEMP = 25.0

Q = 1024
D = 256
K = 100000
BK = 1024
KP = 100352
NB = KP // BK
CH = 128
NCH = KP // CH
CAND = 384
H = 256
NWORK = 32
QPW = Q // NWORK
TP = 80

_NEG = -1e9

_GDN = lax.GatherDimensionNumbers(
    offset_dims=(), collapsed_slice_dims=(0,), start_index_map=(0,))


def _bcast(v, i):
    idx = jnp.full((16, 1), i, jnp.int32)
    return lax.gather(v, idx, _GDN, (1,),
                      mode=lax.GatherScatterMode.PROMISE_IN_BOUNDS)


def _sims_body(q_ref, db_ref, sims_ref, t_ref, cmt_ref, qn_ref, cm_ref):
    j = pl.program_id(0)

    @pl.when(j == 0)
    def _():
        q = q_ref[...]
        qn_ref[...] = q / jnp.maximum(
            jnp.sqrt(jnp.sum(q * q, axis=1, keepdims=True)), 1e-12)

    qn = qn_ref[...]
    db = db_ref[...]
    dbn = db / jnp.maximum(jnp.sqrt(jnp.sum(db * db, axis=1, keepdims=True)), 1e-12)
    sims = lax.dot_general(
        qn, dbn, (((1,), (1,)), ((), ())), preferred_element_type=jnp.float32)
    col = j * BK + lax.broadcasted_iota(jnp.int32, (Q, BK), 1)
    sims = jnp.where(col < K, sims, -2.0)
    sims_ref[...] = sims
    for i in range(BK // CH):
        cm_ref[j * (BK // CH) + i, :] = jnp.max(
            sims[:, i * CH:(i + 1) * CH], axis=1)

    @pl.when(j == NB - 1)
    def _():
        x = cm_ref[...]
        cmt_ref[...] = x.T

        def step(_, x):
            m = jnp.max(x, axis=0, keepdims=True)
            return jnp.where(x >= m, -3.0, x)

        x = lax.fori_loop(0, TOP_K - 1, step, x)
        t_ref[...] = jnp.max(x, axis=0, keepdims=True)


@jax.jit
def _stage_a(query, en_db):
    return pl.pallas_call(
        _sims_body,
        grid=(NB,),
        in_specs=[
            pl.BlockSpec((Q, D), lambda j: (0, 0)),
            pl.BlockSpec((BK, D), lambda j: (j, 0)),
        ],
        out_specs=[
            pl.BlockSpec((Q, BK), lambda j: (0, j)),
            pl.BlockSpec((1, Q), lambda j: (0, 0)),
            pl.BlockSpec((Q, NCH), lambda j: (0, 0)),
        ],
        out_shape=[
            jax.ShapeDtypeStruct((Q, KP), jnp.float32),
            jax.ShapeDtypeStruct((1, Q), jnp.float32),
            jax.ShapeDtypeStruct((Q, NCH), jnp.float32),
        ],
        scratch_shapes=[
            pltpu.VMEM((Q, D), jnp.float32),
            pltpu.VMEM((NCH, Q), jnp.float32),
        ],
    )(query, en_db)


def _tcm_body(cm_ref, t_ref, cmt_ref):
    x = cm_ref[...]
    cmt_ref[...] = x.T

    def step(_, x):
        m = jnp.max(x, axis=0, keepdims=True)
        return jnp.where(x >= m, -3.0, x)

    x = lax.fori_loop(0, TOP_K - 1, step, x)
    t_ref[...] = jnp.max(x, axis=0, keepdims=True)


@jax.jit
def _stage_b(cm):
    return pl.pallas_call(
        _tcm_body,
        out_shape=[
            jax.ShapeDtypeStruct((1, Q), jnp.float32),
            jax.ShapeDtypeStruct((Q, NCH), jnp.float32),
        ],
    )(cm)


def _compact_body(simsc, cmt, tcm, ovals, oidx,
                  tloc, cmloc, hitidx, hitbase, nh, gbuf0, gbuf1,
                  cvals, cidx, semg0, semg1):
    wid = lax.axis_index("s") * 2 + lax.axis_index("c")
    qbase = wid * QPW
    pltpu.sync_copy(tcm.at[0, pl.ds(qbase, QPW)], tloc)
    lane = lax.iota(jnp.int32, 16)

    def build(ql, _):
        qg = qbase + ql
        tq = plsc.load_gather(tloc, [jnp.full((16,), ql, jnp.int32)])
        pltpu.sync_copy(cmt.at[pl.ds(qg, 1)], cmloc)

        def initb(i, _):
            ids = i * 16 + lane
            hitidx[pl.ds(ql * H + i * 16, 16)] = qg * NCH + ids
            return 0

        lax.fori_loop(0, H // 16, initb, 0)

        def vstep(i, off):
            v = cmloc[0, pl.ds(i * 16, 16)]
            m = v >= tq
            c = plsc.cumsum(m.astype(jnp.int32))
            pos = ql * H + jnp.minimum(off + c - 1, H - 1)
            ids = i * 16 + lane
            plsc.store_scatter(hitidx, [pos], qg * NCH + ids, mask=m)
            plsc.store_scatter(hitbase, [pos], ids * CH, mask=m)
            return off + jnp.sum(m.astype(jnp.int32))

        off = lax.fori_loop(0, NCH // 16, vstep, jnp.int32(0))
        nh[ql] = jnp.minimum(off, H)
        return 0

    lax.fori_loop(0, QPW, build, 0)

    pltpu.async_copy(simsc.at[hitidx.at[pl.ds(0, H)]], gbuf0, semg0)

    def scan_query(ql, gbuf):
        qg = qbase + ql
        tq = plsc.load_gather(tloc, [jnp.full((16,), ql, jnp.int32)])

        def initc(i, _):
            cvals[pl.ds(i * 16, 16)] = jnp.full((16,), _NEG, jnp.float32)
            return 0

        lax.fori_loop(0, CAND // 16, initc, 0)
        n = nh[ql]

        def chunk(j, off):
            hb = hitbase[pl.ds(ql * H + (j & ~15), 16)]
            kbase = _bcast(hb, j & 15)
            for r in range(8):
                v = gbuf[j, pl.ds(r * 16, 16)]
                m = v >= tq
                c = plsc.cumsum(m.astype(jnp.int32))
                pos = jnp.minimum(off + c - 1, CAND - 1)
                idxv = kbase + (r * 16 + lane)
                plsc.store_scatter(cvals, [pos], v, mask=m)
                plsc.store_scatter(cidx, [pos], idxv, mask=m)
                off = off + jnp.sum(m.astype(jnp.int32))
            return off

        lax.fori_loop(0, n, chunk, jnp.int32(0))
        pltpu.sync_copy(cvals, ovals.at[qg])
        pltpu.sync_copy(cidx, oidx.at[qg])

    def outer(o, _):
        for b, (gbuf, semg) in enumerate(((gbuf0, semg0), (gbuf1, semg1))):
            ql = o * 2 + b
            ngbuf = gbuf1 if b == 0 else gbuf0
            nsemg = semg1 if b == 0 else semg0

            pltpu.make_async_copy(simsc.at[hitidx.at[pl.ds(ql * H, H)]], gbuf, semg).wait()

            @pl.when(ql + 1 < QPW)
            def _():
                pltpu.async_copy(
                    simsc.at[hitidx.at[pl.ds((ql + 1) * H, H)]], ngbuf, nsemg)

            scan_query(ql, gbuf)
        return 0

    lax.fori_loop(0, QPW // 2, outer, 0)


@jax.jit
def _stage_c(sims, cmt, tcm):
    simsc = jnp.reshape(sims, (Q * NCH, CH))
    mesh = plsc.VectorSubcoreMesh(core_axis_name="c", subcore_axis_name="s")
    return pl.kernel(
        _compact_body,
        out_type=[
            jax.ShapeDtypeStruct((Q, CAND), jnp.float32),
            jax.ShapeDtypeStruct((Q, CAND), jnp.int32),
        ],
        mesh=mesh,
        compiler_params=pltpu.CompilerParams(needs_layout_passes=False),
        scratch_types=[
            pltpu.VMEM((QPW,), jnp.float32),
            pltpu.VMEM((1, NCH), jnp.float32),
            pltpu.VMEM((QPW * H,), jnp.int32),
            pltpu.VMEM((QPW * H,), jnp.int32),
            pltpu.SMEM((QPW,), jnp.int32),
            pltpu.VMEM((H, CH), jnp.float32),
            pltpu.VMEM((H, CH), jnp.float32),
            pltpu.VMEM((CAND,), jnp.float32),
            pltpu.VMEM((CAND,), jnp.int32),
            pltpu.SemaphoreType.DMA,
            pltpu.SemaphoreType.DMA,
        ],
    )(simsc, cmt, tcm)


def _select_body(vals_ref, idx_ref, w_ref, ti_ref):
    V = vals_ref[...]
    I = idx_ref[...]
    iota_c = lax.broadcasted_iota(jnp.int32, (Q, CAND), 1)
    iota_t = lax.broadcasted_iota(jnp.int32, (Q, TP), 1)

    def step(i, carry):
        V, tv, ti = carry
        m = jnp.max(V, axis=1, keepdims=True)
        eq = V >= m
        fidx = jnp.min(jnp.where(eq, iota_c, CAND), axis=1, keepdims=True)
        sel = iota_c == fidx
        selidx = jnp.sum(jnp.where(sel, I, 0), axis=1, keepdims=True)
        V = jnp.where(sel, _NEG, V)
        hit = iota_t == i
        tv = jnp.where(hit, m, tv)
        ti = jnp.where(hit, selidx, ti)
        return V, tv, ti

    tv0 = jnp.full((Q, TP), _NEG, jnp.float32)
    ti0 = jnp.zeros((Q, TP), jnp.int32)
    _, tv, ti = lax.fori_loop(0, TOP_K, step, (V, tv0, ti0))
    m0 = tv[:, 0:1]
    valid = iota_t < TOP_K
    e = jnp.where(valid, jnp.exp((tv - m0) * INV_TEMP), 0.0)
    z = jnp.sum(e, axis=1, keepdims=True)
    w_ref[...] = e / z
    ti_ref[...] = jnp.where(valid, ti, 0)


@jax.jit
def _stage_d(cvals, cidx):
    return pl.pallas_call(
        _select_body,
        out_shape=[
            jax.ShapeDtypeStruct((Q, TP), jnp.float32),
            jax.ShapeDtypeStruct((Q, TP), jnp.int32),
        ],
    )(cvals, cidx)


def _pool_body(ti, w, es, out, idxall, wall, idxb0, idxb1,
               rows0, rows1, obuf, semg0, semg1):
    wid = lax.axis_index("s") * 2 + lax.axis_index("c")
    qbase = wid * QPW
    pltpu.sync_copy(ti.at[pl.ds(qbase, QPW)], idxall)
    pltpu.sync_copy(w.at[pl.ds(qbase, QPW)], wall)

    def fill_idx(ql, idxb):
        for i in range(TP // 16):
            idxb[pl.ds(i * 16, 16)] = idxall[ql, pl.ds(i * 16, 16)]

    fill_idx(0, idxb0)
    pltpu.async_copy(es.at[idxb0], rows0, semg0)

    def pool_query(ql, rows):
        qg = qbase + ql
        for dg in range(4):

            def jbstep(jb, accs):
                wv = wall[ql, pl.ds(jb * 16, 16)]
                for r in range(16):
                    wj = _bcast(wv, r)
                    j = jb * 16 + r
                    accs = tuple(
                        accs[d] + wj * rows[j, pl.ds((dg * 4 + d) * 16, 16)]
                        for d in range(4))
                return accs

            accs = tuple(jnp.zeros((16,), jnp.float32) for _ in range(4))
            accs = lax.fori_loop(0, TP // 16, jbstep, accs)
            for d in range(4):
                obuf[pl.ds((dg * 4 + d) * 16, 16)] = accs[d]
        pltpu.sync_copy(obuf, out.at[qg])

    def outer(o, _):
        for b, (rows, semg) in enumerate(((rows0, semg0), (rows1, semg1))):
            ql = o * 2 + b
            nrows = rows1 if b == 0 else rows0
            nsemg = semg1 if b == 0 else semg0

            idxb = idxb0 if b == 0 else idxb1
            nidxb = idxb1 if b == 0 else idxb0

            pltpu.make_async_copy(es.at[idxb], rows, semg).wait()

            @pl.when(ql + 1 < QPW)
            def _():
                fill_idx(ql + 1, nidxb)
                pltpu.async_copy(es.at[nidxb], nrows, nsemg)

            pool_query(ql, rows)
        return 0

    lax.fori_loop(0, QPW // 2, outer, 0)


@jax.jit
def _stage_e(ti, w, es_db):
    mesh = plsc.VectorSubcoreMesh(core_axis_name="c", subcore_axis_name="s")
    return pl.kernel(
        _pool_body,
        out_type=jax.ShapeDtypeStruct((Q, D), jnp.float32),
        mesh=mesh,
        compiler_params=pltpu.CompilerParams(needs_layout_passes=False),
        scratch_types=[
            pltpu.VMEM((QPW, TP), jnp.int32),
            pltpu.VMEM((QPW, TP), jnp.float32),
            pltpu.VMEM((TP,), jnp.int32),
            pltpu.VMEM((TP,), jnp.int32),
            pltpu.VMEM((TP, D), jnp.float32),
            pltpu.VMEM((TP, D), jnp.float32),
            pltpu.VMEM((D,), jnp.float32),
            pltpu.SemaphoreType.DMA,
            pltpu.SemaphoreType.DMA,
        ],
    )(ti, w, es_db)


def kernel(query, en_db, es_db):
    sims, tcm, cmt = _stage_a(query, en_db)
    cvals, cidx = _stage_c(sims, cmt, tcm)
    w, ti = _stage_d(cvals, cidx)
    return _stage_e(ti, w, es_db)
